# Initial kernel scaffold; baseline (speedup 1.0000x reference)
import jax
import jax.numpy as jnp
from jax.experimental import pallas as pl


def kernel(x, edge_index, W_w, W_b, a_w):
    raise NotImplementedError("write your pallas kernel here")


# speedup vs baseline: 9.2008x; 9.2008x over previous
import jax
import jax.numpy as jnp
from jax import lax
from jax.experimental import pallas as pl
from jax.experimental.pallas import tpu as pltpu
from jax.experimental.pallas import tpu_sc as plsc

N = 10000
E = 320000
D = 128
NOUT = 32
SLOPE = 0.2

NC = 2
NS = 16
LL = 16
NW = NC * NS
EPW = E // NW
CHUNK = 80
NCHUNK = EPW // CHUNK
NGRP = CHUNK // LL


def _uv_body(x_ref, w_ref, b_ref, u_ref, v_ref):
    uv = lax.dot_general(x_ref[...], w_ref[...], (((1,), (0,)), ((), ())),
                         preferred_element_type=jnp.float32)
    u_ref[...] = uv[:, :NOUT] + b_ref[...]
    v_ref[...] = uv[:, NOUT:]


def _make_uv(x, w_cat, b2d):
    blk = 1000
    return pl.pallas_call(
        _uv_body,
        grid=(N // blk,),
        in_specs=[
            pl.BlockSpec((blk, D), lambda i: (i, 0)),
            pl.BlockSpec((D, 2 * NOUT), lambda i: (0, 0)),
            pl.BlockSpec((1, NOUT), lambda i: (0, 0)),
        ],
        out_specs=[
            pl.BlockSpec((blk, NOUT), lambda i: (i, 0)),
            pl.BlockSpec((blk, NOUT), lambda i: (i, 0)),
        ],
        out_shape=[
            jax.ShapeDtypeStruct((N, NOUT), jnp.float32),
            jax.ShapeDtypeStruct((N, NOUT), jnp.float32),
        ],
    )(x, w_cat, b2d)


_MESH = plsc.VectorSubcoreMesh(core_axis_name="c", subcore_axis_name="s",
                               num_cores=NC, num_subcores=NS)


def _edge_body(u_hbm, v_hbm, src_hbm, dst_hbm, arep_hbm,
               ex_hbm, parts_hbm,
               srcb, dstb, gu, gv, exb, arv, zb, shared, sem1, sem2):
    c = lax.axis_index("c")
    s = lax.axis_index("s")
    wid = s * NC + c

    @pl.when(s == 0)
    def _():
        @pl.loop(0, N // LL)
        def _(i):
            zb[pl.ds(i * LL, LL)] = jnp.zeros((LL,), jnp.float32)
        pltpu.sync_copy(zb, shared)

    pltpu.sync_copy(arep_hbm, arv)
    plsc.subcore_barrier()

    iot = lax.iota(jnp.int32, LL)

    @pl.loop(0, NCHUNK)
    def _(j):
        eb = wid * EPW + j * CHUNK
        pltpu.sync_copy(src_hbm.at[pl.ds(eb, CHUNK)], srcb)
        pltpu.sync_copy(dst_hbm.at[pl.ds(eb, CHUNK)], dstb)
        cu = pltpu.async_copy(u_hbm.at[srcb], gu, sem1)
        cv = pltpu.async_copy(v_hbm.at[dstb], gv, sem2)
        cu.wait()
        cv.wait()
        for g in range(NGRP):
            evec = iot + (g * LL)
            acc = jnp.zeros((LL,), jnp.float32)
            for k in range(NOUT):
                kvec = jnp.full((LL,), k, jnp.int32)
                zu = plsc.load_gather(gu, [evec, kvec])
                zv = plsc.load_gather(gv, [evec, kvec])
                z = zu + zv
                l = jnp.maximum(z, z * SLOPE)
                acc = acc + arv[k, :] * l
            exb[pl.ds(g * LL, LL)] = jnp.exp(acc)
        pltpu.sync_copy(exb, ex_hbm.at[pl.ds(eb, CHUNK)])
        pltpu.sync_copy(exb, shared.at[srcb], add=True)

    plsc.subcore_barrier()

    @pl.when(s == 0)
    def _():
        pltpu.sync_copy(shared, parts_hbm.at[c])


def _edge_pass(u, v, src, dst, arep):
    return pl.kernel(
        _edge_body,
        out_type=[
            jax.ShapeDtypeStruct((E,), jnp.float32),
            jax.ShapeDtypeStruct((NC, N), jnp.float32),
        ],
        mesh=_MESH,
        compiler_params=pltpu.CompilerParams(needs_layout_passes=False,
                                             use_tc_tiling_on_sc=False),
        scratch_types=[
            pltpu.VMEM((CHUNK,), jnp.int32),
            pltpu.VMEM((CHUNK,), jnp.int32),
            pltpu.VMEM((CHUNK, NOUT), jnp.float32),
            pltpu.VMEM((CHUNK, NOUT), jnp.float32),
            pltpu.VMEM((CHUNK,), jnp.float32),
            pltpu.VMEM((NOUT, LL), jnp.float32),
            pltpu.VMEM((N,), jnp.float32),
            pltpu.VMEM_SHARED((N,), jnp.float32),
            pltpu.SemaphoreType.DMA,
            pltpu.SemaphoreType.DMA,
        ],
    )(u, v, src, dst, arep)


def _norm_body(ex_hbm, src_hbm, parts_hbm, attn_hbm,
               tab, tmp, srcb, exb, ob):
    c = lax.axis_index("c")
    s = lax.axis_index("s")
    wid = s * NC + c

    pltpu.sync_copy(parts_hbm.at[0], tab)
    pltpu.sync_copy(parts_hbm.at[1], tmp)

    @pl.loop(0, N // LL)
    def _(i):
        sl = pl.ds(i * LL, LL)
        tab[sl] = tab[sl] + tmp[sl]

    @pl.loop(0, NCHUNK)
    def _(j):
        eb = wid * EPW + j * CHUNK
        pltpu.sync_copy(src_hbm.at[pl.ds(eb, CHUNK)], srcb)
        pltpu.sync_copy(ex_hbm.at[pl.ds(eb, CHUNK)], exb)
        for g in range(NGRP):
            sl = pl.ds(g * LL, LL)
            idx = srcb[sl]
            sv = plsc.load_gather(tab, [idx])
            ob[sl] = exb[sl] / sv
        pltpu.sync_copy(ob, attn_hbm.at[pl.ds(eb, CHUNK)])


def _norm_pass(ex, src, parts):
    return pl.kernel(
        _norm_body,
        out_type=jax.ShapeDtypeStruct((E,), jnp.float32),
        mesh=_MESH,
        compiler_params=pltpu.CompilerParams(needs_layout_passes=False,
                                             use_tc_tiling_on_sc=False),
        scratch_types=[
            pltpu.VMEM((N,), jnp.float32),
            pltpu.VMEM((N,), jnp.float32),
            pltpu.VMEM((CHUNK,), jnp.int32),
            pltpu.VMEM((CHUNK,), jnp.float32),
            pltpu.VMEM((CHUNK,), jnp.float32),
        ],
    )(ex, src, parts)


def kernel(x, edge_index, W_w, W_b, a_w):
    src = edge_index[0]
    dst = edge_index[1]
    w_cat = jnp.concatenate([W_w[:, :D].T, W_w[:, D:].T], axis=1)
    b2d = W_b.reshape(1, NOUT)
    arep = jnp.broadcast_to(a_w.reshape(NOUT, 1), (NOUT, LL))
    u, v = _make_uv(x, w_cat, b2d)
    ex, parts = _edge_pass(u, v, src, dst, arep)
    return _norm_pass(ex, src, parts)


# speedup vs baseline: 15.7266x; 1.7093x over previous
import jax
import jax.numpy as jnp
from jax import lax
from jax.experimental import pallas as pl
from jax.experimental.pallas import tpu as pltpu
from jax.experimental.pallas import tpu_sc as plsc

N = 10000
E = 320000
D = 128
NOUT = 32
SLOPE = 0.2

NC = 2
NS = 16
LL = 16
NW = NC * NS
EPW = E // NW
CHUNK = 80
NCHUNK = EPW // CHUNK
NGRP = CHUNK // LL


def _uv_body(x_ref, w_ref, b_ref, u_ref, v_ref):
    uv = lax.dot_general(x_ref[...], w_ref[...], (((1,), (0,)), ((), ())),
                         preferred_element_type=jnp.float32)
    u_ref[...] = uv[:, :NOUT] + b_ref[...]
    v_ref[...] = uv[:, NOUT:]


def _make_uv(x, w_cat, b2d):
    blk = 1000
    return pl.pallas_call(
        _uv_body,
        grid=(N // blk,),
        in_specs=[
            pl.BlockSpec((blk, D), lambda i: (i, 0)),
            pl.BlockSpec((D, 2 * NOUT), lambda i: (0, 0)),
            pl.BlockSpec((1, NOUT), lambda i: (0, 0)),
        ],
        out_specs=[
            pl.BlockSpec((blk, NOUT), lambda i: (i, 0)),
            pl.BlockSpec((blk, NOUT), lambda i: (i, 0)),
        ],
        out_shape=[
            jax.ShapeDtypeStruct((N, NOUT), jnp.float32),
            jax.ShapeDtypeStruct((N, NOUT), jnp.float32),
        ],
    )(x, w_cat, b2d)


_MESH = plsc.VectorSubcoreMesh(core_axis_name="c", subcore_axis_name="s",
                               num_cores=NC, num_subcores=NS)


def _edge_body(u_hbm, v_hbm, src_hbm, dst_hbm, arep_hbm,
               ex_hbm, parts_hbm,
               srcall, dstall, gu, gv, exw, arv, zb, shared, semu, semv):
    c = lax.axis_index("c")
    s = lax.axis_index("s")
    wid = s * NC + c

    @pl.when(s == 0)
    def _():
        @pl.loop(0, N // LL)
        def _(i):
            zb[pl.ds(i * LL, LL)] = jnp.zeros((LL,), jnp.float32)
        pltpu.sync_copy(zb, shared)

    pltpu.sync_copy(arep_hbm, arv)
    pltpu.sync_copy(src_hbm.at[wid], srcall)
    pltpu.sync_copy(dst_hbm.at[wid], dstall)
    plsc.subcore_barrier()

    iot = lax.iota(jnp.int32, LL)

    def issue(j, slot):
        pltpu.async_copy(u_hbm.at[srcall.at[j]], gu.at[slot], semu.at[slot])
        pltpu.async_copy(v_hbm.at[dstall.at[j]], gv.at[slot], semv.at[slot])

    issue(0, 0)

    @pl.loop(0, NCHUNK)
    def _(j):
        par = lax.rem(j, 2)
        gup = gu.at[par]
        gvp = gv.at[par]
        pltpu.make_async_copy(u_hbm.at[srcall.at[j]], gup, semu.at[par]).wait()
        pltpu.make_async_copy(v_hbm.at[dstall.at[j]], gvp, semv.at[par]).wait()

        @pl.when(j + 1 < NCHUNK)
        def _():
            issue(j + 1, 1 - par)

        for g in range(NGRP):
            evec = iot + (g * LL)
            acc = jnp.zeros((LL,), jnp.float32)
            for k in range(NOUT):
                kvec = jnp.full((LL,), k, jnp.int32)
                zu = plsc.load_gather(gup, [evec, kvec])
                zv = plsc.load_gather(gvp, [evec, kvec])
                z = zu + zv
                l = jnp.maximum(z, z * SLOPE)
                acc = acc + arv[k, :] * l
            exw[j, pl.ds(g * LL, LL)] = jnp.exp(acc)
        pltpu.sync_copy(exw.at[j], shared.at[srcall.at[j]], add=True)

    pltpu.sync_copy(exw, ex_hbm.at[wid])
    plsc.subcore_barrier()

    @pl.when(s == 0)
    def _():
        pltpu.sync_copy(shared, parts_hbm.at[c])


def _edge_pass(u, v, src3, dst3, arep):
    return pl.kernel(
        _edge_body,
        out_type=[
            jax.ShapeDtypeStruct((NW, NCHUNK, CHUNK), jnp.float32),
            jax.ShapeDtypeStruct((NC, N), jnp.float32),
        ],
        mesh=_MESH,
        compiler_params=pltpu.CompilerParams(needs_layout_passes=False,
                                             use_tc_tiling_on_sc=False),
        scratch_types=[
            pltpu.VMEM((NCHUNK, CHUNK), jnp.int32),
            pltpu.VMEM((NCHUNK, CHUNK), jnp.int32),
            pltpu.VMEM((2, CHUNK, NOUT), jnp.float32),
            pltpu.VMEM((2, CHUNK, NOUT), jnp.float32),
            pltpu.VMEM((NCHUNK, CHUNK), jnp.float32),
            pltpu.VMEM((NOUT, LL), jnp.float32),
            pltpu.VMEM((N,), jnp.float32),
            pltpu.VMEM_SHARED((N,), jnp.float32),
            pltpu.SemaphoreType.DMA((2,)),
            pltpu.SemaphoreType.DMA((2,)),
        ],
    )(u, v, src3, dst3, arep)


def _norm_body(ex_hbm, src_hbm, parts_hbm, attn_hbm,
               tab, tmp, srcall, exall, oall):
    c = lax.axis_index("c")
    s = lax.axis_index("s")
    wid = s * NC + c

    pltpu.sync_copy(parts_hbm.at[0], tab)
    pltpu.sync_copy(src_hbm.at[wid], srcall)
    pltpu.sync_copy(ex_hbm.at[wid], exall)
    pltpu.sync_copy(parts_hbm.at[1], tmp)

    @pl.loop(0, N // LL)
    def _(i):
        sl = pl.ds(i * LL, LL)
        tab[sl] = tab[sl] + tmp[sl]

    @pl.loop(0, NCHUNK)
    def _(j):
        for g in range(NGRP):
            sl = pl.ds(g * LL, LL)
            idx = srcall[j, sl]
            sv = plsc.load_gather(tab, [idx])
            oall[j, sl] = exall[j, sl] / sv

    pltpu.sync_copy(oall, attn_hbm.at[wid])


def _norm_pass(ex3, src3, parts):
    return pl.kernel(
        _norm_body,
        out_type=jax.ShapeDtypeStruct((NW, NCHUNK, CHUNK), jnp.float32),
        mesh=_MESH,
        compiler_params=pltpu.CompilerParams(needs_layout_passes=False,
                                             use_tc_tiling_on_sc=False),
        scratch_types=[
            pltpu.VMEM((N,), jnp.float32),
            pltpu.VMEM((N,), jnp.float32),
            pltpu.VMEM((NCHUNK, CHUNK), jnp.int32),
            pltpu.VMEM((NCHUNK, CHUNK), jnp.float32),
            pltpu.VMEM((NCHUNK, CHUNK), jnp.float32),
        ],
    )(ex3, src3, parts)


def kernel(x, edge_index, W_w, W_b, a_w):
    src3 = edge_index[0].reshape(NW, NCHUNK, CHUNK)
    dst3 = edge_index[1].reshape(NW, NCHUNK, CHUNK)
    w_cat = jnp.concatenate([W_w[:, :D].T, W_w[:, D:].T], axis=1)
    b2d = W_b.reshape(1, NOUT)
    arep = jnp.broadcast_to(a_w.reshape(NOUT, 1), (NOUT, LL))
    u, v = _make_uv(x, w_cat, b2d)
    ex3, parts = _edge_pass(u, v, src3, dst3, arep)
    return _norm_pass(ex3, src3, parts).reshape(E)


# speedup vs baseline: 16.4642x; 1.0469x over previous
import jax
import jax.numpy as jnp
from jax import lax
from jax.experimental import pallas as pl
from jax.experimental.pallas import tpu as pltpu
from jax.experimental.pallas import tpu_sc as plsc

N = 10000
E = 320000
D = 128
NOUT = 32
SLOPE = 0.2

NC = 2
NS = 16
LL = 16
NW = NC * NS
EPW = E // NW
CHUNK = 80
NCHUNK = EPW // CHUNK
NGRP = CHUNK // LL


def _uv_body(x_ref, w_ref, b_ref, u_ref, v_ref):
    uv = lax.dot_general(x_ref[...], w_ref[...], (((1,), (0,)), ((), ())),
                         preferred_element_type=jnp.float32)
    u_ref[...] = uv[:, :NOUT] + b_ref[...]
    v_ref[...] = uv[:, NOUT:]


def _make_uv(x, w_cat, b2d):
    blk = 1000
    return pl.pallas_call(
        _uv_body,
        grid=(N // blk,),
        in_specs=[
            pl.BlockSpec((blk, D), lambda i: (i, 0)),
            pl.BlockSpec((D, 2 * NOUT), lambda i: (0, 0)),
            pl.BlockSpec((1, NOUT), lambda i: (0, 0)),
        ],
        out_specs=[
            pl.BlockSpec((blk, NOUT), lambda i: (i, 0)),
            pl.BlockSpec((blk, NOUT), lambda i: (i, 0)),
        ],
        out_shape=[
            jax.ShapeDtypeStruct((N, NOUT), jnp.float32),
            jax.ShapeDtypeStruct((N, NOUT), jnp.float32),
        ],
    )(x, w_cat, b2d)


_MESH = plsc.VectorSubcoreMesh(core_axis_name="c", subcore_axis_name="s",
                               num_cores=NC, num_subcores=NS)


def _edge_body(u_hbm, v_hbm, src_hbm, dst_hbm, arep_hbm,
               ex_hbm, parts_hbm,
               srcall, dstall, gu, gv, exw, arv, zb, shared, semu, semv):
    c = lax.axis_index("c")
    s = lax.axis_index("s")
    wid = s * NC + c

    @pl.when(s == 0)
    def _():
        @pl.loop(0, N // LL)
        def _(i):
            zb[pl.ds(i * LL, LL)] = jnp.zeros((LL,), jnp.float32)
        pltpu.sync_copy(zb, shared)

    pltpu.sync_copy(arep_hbm, arv)
    pltpu.sync_copy(src_hbm.at[wid], srcall)
    pltpu.sync_copy(dst_hbm.at[wid], dstall)
    plsc.subcore_barrier()

    iot = lax.iota(jnp.int32, LL)
    avals = [arv[k, :] for k in range(NOUT)]

    def issue(j, slot):
        pltpu.async_copy(u_hbm.at[srcall.at[j]], gu.at[slot], semu.at[slot])
        pltpu.async_copy(v_hbm.at[dstall.at[j]], gv.at[slot], semv.at[slot])

    issue(0, 0)

    @pl.loop(0, NCHUNK)
    def _(j):
        par = lax.rem(j, 2)
        gup = gu.at[par]
        gvp = gv.at[par]
        pltpu.make_async_copy(u_hbm.at[srcall.at[j]], gup, semu.at[par]).wait()
        pltpu.make_async_copy(v_hbm.at[dstall.at[j]], gvp, semv.at[par]).wait()

        @pl.when(j + 1 < NCHUNK)
        def _():
            issue(j + 1, 1 - par)

        for g in range(NGRP):
            evec = iot + (g * LL)
            acc = jnp.zeros((LL,), jnp.float32)
            for k in range(NOUT):
                kvec = jnp.full((LL,), k, jnp.int32)
                zu = plsc.load_gather(gup, [evec, kvec])
                zv = plsc.load_gather(gvp, [evec, kvec])
                z = zu + zv
                l = jnp.maximum(z, z * SLOPE)
                acc = acc + avals[k] * l
            exw[j, pl.ds(g * LL, LL)] = jnp.exp(acc)
        pltpu.sync_copy(exw.at[j], shared.at[srcall.at[j]], add=True)

    pltpu.sync_copy(exw, ex_hbm.at[wid])
    plsc.subcore_barrier()

    @pl.when(s == 0)
    def _():
        pltpu.sync_copy(shared, parts_hbm.at[c])


def _edge_pass(u, v, src3, dst3, arep):
    return pl.kernel(
        _edge_body,
        out_type=[
            jax.ShapeDtypeStruct((NW, NCHUNK, CHUNK), jnp.float32),
            jax.ShapeDtypeStruct((NC, N), jnp.float32),
        ],
        mesh=_MESH,
        compiler_params=pltpu.CompilerParams(needs_layout_passes=False,
                                             use_tc_tiling_on_sc=False),
        scratch_types=[
            pltpu.VMEM((NCHUNK, CHUNK), jnp.int32),
            pltpu.VMEM((NCHUNK, CHUNK), jnp.int32),
            pltpu.VMEM((2, CHUNK, NOUT), jnp.float32),
            pltpu.VMEM((2, CHUNK, NOUT), jnp.float32),
            pltpu.VMEM((NCHUNK, CHUNK), jnp.float32),
            pltpu.VMEM((NOUT, LL), jnp.float32),
            pltpu.VMEM((N,), jnp.float32),
            pltpu.VMEM_SHARED((N,), jnp.float32),
            pltpu.SemaphoreType.DMA((2,)),
            pltpu.SemaphoreType.DMA((2,)),
        ],
    )(u, v, src3, dst3, arep)


def _norm_body(ex_hbm, src_hbm, parts_hbm, attn_hbm,
               tab, tmp, srcall, exall, oall, s0, s1, s2, s3):
    c = lax.axis_index("c")
    s = lax.axis_index("s")
    wid = s * NC + c

    c0 = pltpu.async_copy(parts_hbm.at[0], tab, s0)
    c1 = pltpu.async_copy(parts_hbm.at[1], tmp, s1)
    c2 = pltpu.async_copy(src_hbm.at[wid], srcall, s2)
    c3 = pltpu.async_copy(ex_hbm.at[wid], exall, s3)
    c0.wait()
    c1.wait()

    @pl.loop(0, N // LL)
    def _(i):
        sl = pl.ds(i * LL, LL)
        tab[sl] = tab[sl] + tmp[sl]

    c2.wait()
    c3.wait()

    @pl.loop(0, NCHUNK)
    def _(j):
        for g in range(NGRP):
            sl = pl.ds(g * LL, LL)
            idx = srcall[j, sl]
            sv = plsc.load_gather(tab, [idx])
            oall[j, sl] = exall[j, sl] / sv

    pltpu.sync_copy(oall, attn_hbm.at[wid])


def _norm_pass(ex3, src3, parts):
    return pl.kernel(
        _norm_body,
        out_type=jax.ShapeDtypeStruct((NW, NCHUNK, CHUNK), jnp.float32),
        mesh=_MESH,
        compiler_params=pltpu.CompilerParams(needs_layout_passes=False,
                                             use_tc_tiling_on_sc=False),
        scratch_types=[
            pltpu.VMEM((N,), jnp.float32),
            pltpu.VMEM((N,), jnp.float32),
            pltpu.VMEM((NCHUNK, CHUNK), jnp.int32),
            pltpu.VMEM((NCHUNK, CHUNK), jnp.float32),
            pltpu.VMEM((NCHUNK, CHUNK), jnp.float32),
            pltpu.SemaphoreType.DMA,
            pltpu.SemaphoreType.DMA,
            pltpu.SemaphoreType.DMA,
            pltpu.SemaphoreType.DMA,
        ],
    )(ex3, src3, parts)


def kernel(x, edge_index, W_w, W_b, a_w):
    src3 = edge_index[0].reshape(NW, NCHUNK, CHUNK)
    dst3 = edge_index[1].reshape(NW, NCHUNK, CHUNK)
    w_cat = jnp.concatenate([W_w[:, :D].T, W_w[:, D:].T], axis=1)
    b2d = W_b.reshape(1, NOUT)
    arep = jnp.broadcast_to(a_w.reshape(NOUT, 1), (NOUT, LL))
    u, v = _make_uv(x, w_cat, b2d)
    ex3, parts = _edge_pass(u, v, src3, dst3, arep)
    return _norm_pass(ex3, src3, parts).reshape(E)


# speedup vs baseline: 17.0247x; 1.0340x over previous
import jax
import jax.numpy as jnp
from jax import lax
from jax.experimental import pallas as pl
from jax.experimental.pallas import tpu as pltpu
from jax.experimental.pallas import tpu_sc as plsc

N = 10000
E = 320000
D = 128
NOUT = 32
SLOPE = 0.2

NC = 2
NS = 16
LL = 16
NW = NC * NS
EPW = E // NW
CHUNK = 400
NCHUNK = EPW // CHUNK
NGRP = CHUNK // LL


def _uv_body(x_ref, w_ref, b_ref, u_ref, v_ref):
    uv = lax.dot_general(x_ref[...], w_ref[...], (((1,), (0,)), ((), ())),
                         preferred_element_type=jnp.float32)
    u_ref[...] = uv[:, :NOUT] + b_ref[...]
    v_ref[...] = uv[:, NOUT:]


def _make_uv(x, w_cat, b2d):
    blk = 1000
    return pl.pallas_call(
        _uv_body,
        grid=(N // blk,),
        in_specs=[
            pl.BlockSpec((blk, D), lambda i: (i, 0)),
            pl.BlockSpec((D, 2 * NOUT), lambda i: (0, 0)),
            pl.BlockSpec((1, NOUT), lambda i: (0, 0)),
        ],
        out_specs=[
            pl.BlockSpec((blk, NOUT), lambda i: (i, 0)),
            pl.BlockSpec((blk, NOUT), lambda i: (i, 0)),
        ],
        out_shape=[
            jax.ShapeDtypeStruct((N, NOUT), jnp.float32),
            jax.ShapeDtypeStruct((N, NOUT), jnp.float32),
        ],
    )(x, w_cat, b2d)


_MESH = plsc.VectorSubcoreMesh(core_axis_name="c", subcore_axis_name="s",
                               num_cores=NC, num_subcores=NS)


def _edge_body(u_hbm, v_hbm, src_hbm, dst_hbm, arep_hbm,
               ex_hbm, parts_hbm,
               srcall, dstall, gu, gv, exw, arv, zb, shared, semu, semv):
    c = lax.axis_index("c")
    s = lax.axis_index("s")
    wid = s * NC + c

    @pl.when(s == 0)
    def _():
        @pl.loop(0, N // LL)
        def _(i):
            zb[pl.ds(i * LL, LL)] = jnp.zeros((LL,), jnp.float32)
        pltpu.sync_copy(zb, shared)

    pltpu.sync_copy(arep_hbm, arv)
    pltpu.sync_copy(src_hbm.at[wid], srcall)
    pltpu.sync_copy(dst_hbm.at[wid], dstall)
    plsc.subcore_barrier()

    iot = lax.iota(jnp.int32, LL)
    avals = [arv[k, :] for k in range(NOUT)]

    def issue(j, slot):
        pltpu.async_copy(u_hbm.at[srcall.at[j]], gu.at[slot], semu.at[slot])
        pltpu.async_copy(v_hbm.at[dstall.at[j]], gv.at[slot], semv.at[slot])

    issue(0, 0)

    @pl.loop(0, NCHUNK)
    def _(j):
        par = lax.rem(j, 2)
        gup = gu.at[par]
        gvp = gv.at[par]
        pltpu.make_async_copy(u_hbm.at[srcall.at[j]], gup, semu.at[par]).wait()
        pltpu.make_async_copy(v_hbm.at[dstall.at[j]], gvp, semv.at[par]).wait()

        @pl.when(j + 1 < NCHUNK)
        def _():
            issue(j + 1, 1 - par)

        @pl.loop(0, NGRP)
        def _(g):
            evec = iot + g * LL
            acc = jnp.zeros((LL,), jnp.float32)
            for k in range(NOUT):
                kvec = jnp.full((LL,), k, jnp.int32)
                zu = plsc.load_gather(gup, [evec, kvec])
                zv = plsc.load_gather(gvp, [evec, kvec])
                z = zu + zv
                l = jnp.maximum(z, z * SLOPE)
                acc = acc + avals[k] * l
            exw[j, pl.ds(g * LL, LL)] = jnp.exp(acc)
        pltpu.sync_copy(exw.at[j], shared.at[srcall.at[j]], add=True)

    pltpu.sync_copy(exw, ex_hbm.at[wid])
    plsc.subcore_barrier()

    @pl.when(s == 0)
    def _():
        pltpu.sync_copy(shared, parts_hbm.at[c])


def _edge_pass(u, v, src3, dst3, arep):
    return pl.kernel(
        _edge_body,
        out_type=[
            jax.ShapeDtypeStruct((NW, NCHUNK, CHUNK), jnp.float32),
            jax.ShapeDtypeStruct((NC, N), jnp.float32),
        ],
        mesh=_MESH,
        compiler_params=pltpu.CompilerParams(needs_layout_passes=False,
                                             use_tc_tiling_on_sc=False),
        scratch_types=[
            pltpu.VMEM((NCHUNK, CHUNK), jnp.int32),
            pltpu.VMEM((NCHUNK, CHUNK), jnp.int32),
            pltpu.VMEM((2, CHUNK, NOUT), jnp.float32),
            pltpu.VMEM((2, CHUNK, NOUT), jnp.float32),
            pltpu.VMEM((NCHUNK, CHUNK), jnp.float32),
            pltpu.VMEM((NOUT, LL), jnp.float32),
            pltpu.VMEM((N,), jnp.float32),
            pltpu.VMEM_SHARED((N,), jnp.float32),
            pltpu.SemaphoreType.DMA((2,)),
            pltpu.SemaphoreType.DMA((2,)),
        ],
    )(u, v, src3, dst3, arep)


def _norm_body(ex_hbm, src_hbm, parts_hbm, attn_hbm,
               tab, tmp, srcall, exall, oall, s0, s1, s2, s3):
    c = lax.axis_index("c")
    s = lax.axis_index("s")
    wid = s * NC + c

    c0 = pltpu.async_copy(parts_hbm.at[0], tab, s0)
    c1 = pltpu.async_copy(parts_hbm.at[1], tmp, s1)
    c2 = pltpu.async_copy(src_hbm.at[wid], srcall, s2)
    c3 = pltpu.async_copy(ex_hbm.at[wid], exall, s3)
    c0.wait()
    c1.wait()

    @pl.loop(0, N // LL)
    def _(i):
        sl = pl.ds(i * LL, LL)
        tab[sl] = tab[sl] + tmp[sl]

    c2.wait()
    c3.wait()

    @pl.loop(0, NCHUNK)
    def _(j):
        for g in range(NGRP):
            sl = pl.ds(g * LL, LL)
            idx = srcall[j, sl]
            sv = plsc.load_gather(tab, [idx])
            oall[j, sl] = exall[j, sl] / sv

    pltpu.sync_copy(oall, attn_hbm.at[wid])


def _norm_pass(ex3, src3, parts):
    return pl.kernel(
        _norm_body,
        out_type=jax.ShapeDtypeStruct((NW, NCHUNK, CHUNK), jnp.float32),
        mesh=_MESH,
        compiler_params=pltpu.CompilerParams(needs_layout_passes=False,
                                             use_tc_tiling_on_sc=False),
        scratch_types=[
            pltpu.VMEM((N,), jnp.float32),
            pltpu.VMEM((N,), jnp.float32),
            pltpu.VMEM((NCHUNK, CHUNK), jnp.int32),
            pltpu.VMEM((NCHUNK, CHUNK), jnp.float32),
            pltpu.VMEM((NCHUNK, CHUNK), jnp.float32),
            pltpu.SemaphoreType.DMA,
            pltpu.SemaphoreType.DMA,
            pltpu.SemaphoreType.DMA,
            pltpu.SemaphoreType.DMA,
        ],
    )(ex3, src3, parts)


def kernel(x, edge_index, W_w, W_b, a_w):
    src3 = edge_index[0].reshape(NW, NCHUNK, CHUNK)
    dst3 = edge_index[1].reshape(NW, NCHUNK, CHUNK)
    w_cat = jnp.concatenate([W_w[:, :D].T, W_w[:, D:].T], axis=1)
    b2d = W_b.reshape(1, NOUT)
    arep = jnp.broadcast_to(a_w.reshape(NOUT, 1), (NOUT, LL))
    u, v = _make_uv(x, w_cat, b2d)
    ex3, parts = _edge_pass(u, v, src3, dst3, arep)
    return _norm_pass(ex3, src3, parts).reshape(E)


# speedup vs baseline: 17.2573x; 1.0137x over previous
import jax
import jax.numpy as jnp
from jax import lax
from jax.experimental import pallas as pl
from jax.experimental.pallas import tpu as pltpu
from jax.experimental.pallas import tpu_sc as plsc

N = 10000
E = 320000
D = 128
NOUT = 32
SLOPE = 0.2

NC = 2
NS = 16
LL = 16
NW = NC * NS
EPW = E // NW
CHUNK = 400
NCHUNK = EPW // CHUNK
NGRP = CHUNK // LL


def _uv_body(x_ref, w_ref, b_ref, u_ref, v_ref):
    uv = lax.dot_general(x_ref[...], w_ref[...], (((1,), (0,)), ((), ())),
                         preferred_element_type=jnp.float32)
    u_ref[...] = uv[:, :NOUT] + b_ref[...]
    v_ref[...] = uv[:, NOUT:]


def _make_uv(x, w_cat, b2d):
    blk = 1000
    return pl.pallas_call(
        _uv_body,
        grid=(N // blk,),
        in_specs=[
            pl.BlockSpec((blk, D), lambda i: (i, 0)),
            pl.BlockSpec((D, 2 * NOUT), lambda i: (0, 0)),
            pl.BlockSpec((1, NOUT), lambda i: (0, 0)),
        ],
        out_specs=[
            pl.BlockSpec((blk, NOUT), lambda i: (i, 0)),
            pl.BlockSpec((blk, NOUT), lambda i: (i, 0)),
        ],
        out_shape=[
            jax.ShapeDtypeStruct((N, NOUT), jnp.float32),
            jax.ShapeDtypeStruct((N, NOUT), jnp.float32),
        ],
    )(x, w_cat, b2d)


_MESH = plsc.VectorSubcoreMesh(core_axis_name="c", subcore_axis_name="s",
                               num_cores=NC, num_subcores=NS)


def _edge_body(u_hbm, v_hbm, src_hbm, dst_hbm, arep_hbm,
               ex_hbm, parts_hbm,
               srcall, dstall, gu, gv, exw, arv, zb, shared, su, sv,
               semu, semv, semsc):
    c = lax.axis_index("c")
    s = lax.axis_index("s")
    wid = s * NC + c

    rpt = N // NS
    cu0 = pltpu.async_copy(u_hbm.at[pl.ds(s * rpt, rpt)],
                           su.at[pl.ds(s * rpt, rpt)], semu.at[0])
    cv0 = pltpu.async_copy(v_hbm.at[pl.ds(s * rpt, rpt)],
                           sv.at[pl.ds(s * rpt, rpt)], semv.at[0])

    @pl.when(s == 0)
    def _():
        @pl.loop(0, CHUNK // LL)
        def _(i):
            zb[pl.ds(i * LL, LL)] = jnp.zeros((LL,), jnp.float32)

        @pl.loop(0, N // CHUNK)
        def _(i):
            pltpu.sync_copy(zb, shared.at[pl.ds(i * CHUNK, CHUNK)])

    pltpu.sync_copy(arep_hbm, arv)
    pltpu.sync_copy(src_hbm.at[wid], srcall)
    pltpu.sync_copy(dst_hbm.at[wid], dstall)
    cu0.wait()
    cv0.wait()
    plsc.subcore_barrier()

    iot = lax.iota(jnp.int32, LL)
    avals = [arv[k, :] for k in range(NOUT)]

    def issue(j, slot):
        pltpu.async_copy(su.at[srcall.at[j]], gu.at[slot], semu.at[slot])
        pltpu.async_copy(sv.at[dstall.at[j]], gv.at[slot], semv.at[slot])

    issue(0, 0)

    @pl.loop(0, NCHUNK)
    def _(j):
        par = lax.rem(j, 2)
        gup = gu.at[par]
        gvp = gv.at[par]
        pltpu.make_async_copy(su.at[srcall.at[j]], gup, semu.at[par]).wait()
        pltpu.make_async_copy(sv.at[dstall.at[j]], gvp, semv.at[par]).wait()

        @pl.when(j + 1 < NCHUNK)
        def _():
            issue(j + 1, 1 - par)

        @pl.loop(0, NGRP)
        def _(g):
            evec = iot + g * LL
            acc = jnp.zeros((LL,), jnp.float32)
            for k in range(NOUT):
                kvec = jnp.full((LL,), k, jnp.int32)
                zu = plsc.load_gather(gup, [evec, kvec])
                zv = plsc.load_gather(gvp, [evec, kvec])
                z = zu + zv
                l = jnp.maximum(z, z * SLOPE)
                acc = acc + avals[k] * l
            exw[j, pl.ds(g * LL, LL)] = jnp.exp(acc)
        pltpu.async_copy(exw.at[j], shared.at[srcall.at[j]], semsc, add=True)

        @pl.when(j >= 2)
        def _():
            jm = j - 2
            pltpu.make_async_copy(exw.at[jm], shared.at[srcall.at[jm]],
                                  semsc).wait()

    @pl.loop(NCHUNK - 2, NCHUNK)
    def _(j):
        pltpu.make_async_copy(exw.at[j], shared.at[srcall.at[j]], semsc).wait()

    pltpu.sync_copy(exw, ex_hbm.at[wid])
    plsc.subcore_barrier()

    @pl.when(s == 0)
    def _():
        pltpu.sync_copy(shared, parts_hbm.at[c])


def _edge_pass(u, v, src3, dst3, arep):
    return pl.kernel(
        _edge_body,
        out_type=[
            jax.ShapeDtypeStruct((NW, NCHUNK, CHUNK), jnp.float32),
            jax.ShapeDtypeStruct((NC, N), jnp.float32),
        ],
        mesh=_MESH,
        compiler_params=pltpu.CompilerParams(needs_layout_passes=False,
                                             use_tc_tiling_on_sc=False),
        scratch_types=[
            pltpu.VMEM((NCHUNK, CHUNK), jnp.int32),
            pltpu.VMEM((NCHUNK, CHUNK), jnp.int32),
            pltpu.VMEM((2, CHUNK, NOUT), jnp.float32),
            pltpu.VMEM((2, CHUNK, NOUT), jnp.float32),
            pltpu.VMEM((NCHUNK, CHUNK), jnp.float32),
            pltpu.VMEM((NOUT, LL), jnp.float32),
            pltpu.VMEM((CHUNK,), jnp.float32),
            pltpu.VMEM_SHARED((N,), jnp.float32),
            pltpu.VMEM_SHARED((N, NOUT), jnp.float32),
            pltpu.VMEM_SHARED((N, NOUT), jnp.float32),
            pltpu.SemaphoreType.DMA((2,)),
            pltpu.SemaphoreType.DMA((2,)),
            pltpu.SemaphoreType.DMA,
        ],
    )(u, v, src3, dst3, arep)


def _norm_body(ex_hbm, src_hbm, parts_hbm, attn_hbm,
               tab, tmp, srcall, exall, oall, s0, s1, s2, s3):
    c = lax.axis_index("c")
    s = lax.axis_index("s")
    wid = s * NC + c

    c0 = pltpu.async_copy(parts_hbm.at[0], tab, s0)
    c1 = pltpu.async_copy(parts_hbm.at[1], tmp, s1)
    c2 = pltpu.async_copy(src_hbm.at[wid], srcall, s2)
    c3 = pltpu.async_copy(ex_hbm.at[wid], exall, s3)
    c0.wait()
    c1.wait()

    @pl.loop(0, N // LL)
    def _(i):
        sl = pl.ds(i * LL, LL)
        tab[sl] = tab[sl] + tmp[sl]

    c2.wait()
    c3.wait()

    @pl.loop(0, NCHUNK)
    def _(j):
        for g in range(NGRP):
            sl = pl.ds(g * LL, LL)
            idx = srcall[j, sl]
            sv = plsc.load_gather(tab, [idx])
            oall[j, sl] = exall[j, sl] / sv

    pltpu.sync_copy(oall, attn_hbm.at[wid])


def _norm_pass(ex3, src3, parts):
    return pl.kernel(
        _norm_body,
        out_type=jax.ShapeDtypeStruct((NW, NCHUNK, CHUNK), jnp.float32),
        mesh=_MESH,
        compiler_params=pltpu.CompilerParams(needs_layout_passes=False,
                                             use_tc_tiling_on_sc=False),
        scratch_types=[
            pltpu.VMEM((N,), jnp.float32),
            pltpu.VMEM((N,), jnp.float32),
            pltpu.VMEM((NCHUNK, CHUNK), jnp.int32),
            pltpu.VMEM((NCHUNK, CHUNK), jnp.float32),
            pltpu.VMEM((NCHUNK, CHUNK), jnp.float32),
            pltpu.SemaphoreType.DMA,
            pltpu.SemaphoreType.DMA,
            pltpu.SemaphoreType.DMA,
            pltpu.SemaphoreType.DMA,
        ],
    )(ex3, src3, parts)


def kernel(x, edge_index, W_w, W_b, a_w):
    src3 = edge_index[0].reshape(NW, NCHUNK, CHUNK)
    dst3 = edge_index[1].reshape(NW, NCHUNK, CHUNK)
    w_cat = jnp.concatenate([W_w[:, :D].T, W_w[:, D:].T], axis=1)
    b2d = W_b.reshape(1, NOUT)
    arep = jnp.broadcast_to(a_w.reshape(NOUT, 1), (NOUT, LL))
    u, v = _make_uv(x, w_cat, b2d)
    ex3, parts = _edge_pass(u, v, src3, dst3, arep)
    return _norm_pass(ex3, src3, parts).reshape(E)
